# Initial kernel scaffold; baseline (speedup 1.0000x reference)
import jax
import jax.numpy as jnp
from jax.experimental import pallas as pl


def kernel(x, pos_embedding):
    raise NotImplementedError("write your pallas kernel here")


# speedup vs baseline: 2.4112x; 2.4112x over previous
import jax
import jax.numpy as jnp
from jax.experimental import pallas as pl


_BL = 512


def _add_kernel(x_ref, pe_ref, o_ref):
    o_ref[...] = x_ref[...] + pe_ref[...]


def kernel(x, pos_embedding):
    if x.ndim != 3:
        raise ValueError(
            f'Expected input to have 3 dimensions, but got {x.ndim} dimensions')
    B, L, D = x.shape
    pe = pos_embedding[:L]
    grid = (B, L // _BL)
    return pl.pallas_call(
        _add_kernel,
        grid=grid,
        in_specs=[
            pl.BlockSpec((1, _BL, D), lambda b, l: (b, l, 0)),
            pl.BlockSpec((_BL, D), lambda b, l: (l, 0)),
        ],
        out_specs=pl.BlockSpec((1, _BL, D), lambda b, l: (b, l, 0)),
        out_shape=jax.ShapeDtypeStruct((B, L, D), x.dtype),
    )(x, pe)


# speedup vs baseline: 2.8579x; 1.1852x over previous
import jax
import jax.numpy as jnp
from jax.experimental import pallas as pl


_BL = 512


def _add_kernel(x_ref, pe_ref, o_ref):
    o_ref[...] = x_ref[...] + pe_ref[...]


def kernel(x, pos_embedding):
    if x.ndim != 3:
        raise ValueError(
            f'Expected input to have 3 dimensions, but got {x.ndim} dimensions')
    B, L, D = x.shape
    pe = pos_embedding[:L]
    grid = (L // _BL, B)
    return pl.pallas_call(
        _add_kernel,
        grid=grid,
        in_specs=[
            pl.BlockSpec((1, _BL, D), lambda l, b: (b, l, 0)),
            pl.BlockSpec((_BL, D), lambda l, b: (l, 0)),
        ],
        out_specs=pl.BlockSpec((1, _BL, D), lambda l, b: (b, l, 0)),
        out_shape=jax.ShapeDtypeStruct((B, L, D), x.dtype),
    )(x, pe)
